# Initial kernel scaffold; baseline (speedup 1.0000x reference)
import jax
import jax.numpy as jnp
from jax.experimental import pallas as pl


def kernel(x, embedding):
    raise NotImplementedError("write your pallas kernel here")


# speedup vs baseline: 5.7519x; 5.7519x over previous
import functools

import jax
import jax.numpy as jnp
from jax import lax
from jax.experimental import pallas as pl
from jax.experimental.pallas import tpu as pltpu
from jax.experimental.pallas import tpu_sc as plsc

B, S, H = 1024, 200, 128
N = B * S
NUM_WORKERS = 32
ROWS_PER_W = N // NUM_WORKERS
CHUNK = 128
N_CHUNKS = ROWS_PER_W // CHUNK

_mesh = plsc.VectorSubcoreMesh(core_axis_name="c", subcore_axis_name="s")


@functools.partial(
    pl.kernel,
    mesh=_mesh,
    out_type=jax.ShapeDtypeStruct((N, H), jnp.float32),
    scratch_types=[
        pltpu.VMEM((N_CHUNKS, CHUNK), jnp.int32),
        pltpu.VMEM((CHUNK, H), jnp.float32),
        pltpu.SemaphoreType.DMA,
    ],
)
def _gather_kernel(idx_hbm, table_hbm, out_hbm, idx_v, rows_v, sem):
    wid = lax.axis_index("s") * 2 + lax.axis_index("c")
    base = wid * ROWS_PER_W
    pltpu.sync_copy(idx_hbm.at[wid], idx_v)

    def body(c, _):
        pltpu.async_copy(table_hbm.at[idx_v.at[c]], rows_v, sem).wait()
        pltpu.sync_copy(rows_v, out_hbm.at[pl.ds(base + c * CHUNK, CHUNK)])
        return ()

    lax.fori_loop(0, N_CHUNKS, body, (), unroll=False)


def kernel(x, embedding):
    idx = x.reshape(NUM_WORKERS, N_CHUNKS, CHUNK)
    out = _gather_kernel(idx, embedding)
    return out.reshape(B, S, H)


# speedup vs baseline: 7.3125x; 1.2713x over previous
import functools

import jax
import jax.numpy as jnp
from jax import lax
from jax.experimental import pallas as pl
from jax.experimental.pallas import tpu as pltpu
from jax.experimental.pallas import tpu_sc as plsc

B, S, H = 1024, 200, 128
N = B * S
NUM_WORKERS = 32
ROWS_PER_W = N // NUM_WORKERS
CHUNK = 128
N_CHUNKS = ROWS_PER_W // CHUNK

_mesh = plsc.VectorSubcoreMesh(core_axis_name="c", subcore_axis_name="s")


@functools.partial(
    pl.kernel,
    mesh=_mesh,
    out_type=jax.ShapeDtypeStruct((N, H), jnp.float32),
    scratch_types=[
        pltpu.VMEM((N_CHUNKS, CHUNK), jnp.int32),
        pltpu.VMEM((CHUNK, H), jnp.float32),
        pltpu.VMEM((CHUNK, H), jnp.float32),
        pltpu.SemaphoreType.DMA,
        pltpu.SemaphoreType.DMA,
        pltpu.SemaphoreType.DMA,
        pltpu.SemaphoreType.DMA,
    ],
)
def _gather_kernel(idx_hbm, table_hbm, out_hbm, idx_v, buf0, buf1,
                   gs0, gs1, ws0, ws1):
    wid = lax.axis_index("s") * 2 + lax.axis_index("c")
    base = wid * ROWS_PER_W
    pltpu.sync_copy(idx_hbm.at[wid], idx_v)

    def gather(c, buf, sem):
        return pltpu.async_copy(table_hbm.at[idx_v.at[c]], buf, sem)

    def write(c, buf, sem):
        return pltpu.async_copy(buf, out_hbm.at[pl.ds(base + c * CHUNK, CHUNK)], sem)

    gather(0, buf0, gs0)
    gather(1, buf1, gs1)

    def body(i, _):
        c0 = 2 * i
        c1 = c0 + 1
        pltpu.make_async_copy(table_hbm.at[idx_v.at[c0]], buf0, gs0).wait()
        write(c0, buf0, ws0)
        pltpu.make_async_copy(table_hbm.at[idx_v.at[c1]], buf1, gs1).wait()
        write(c1, buf1, ws1)

        @pl.when(c0 + 2 < N_CHUNKS)
        def _():
            pltpu.make_async_copy(
                buf0, out_hbm.at[pl.ds(base + c0 * CHUNK, CHUNK)], ws0).wait()
            gather(c0 + 2, buf0, gs0)
            pltpu.make_async_copy(
                buf1, out_hbm.at[pl.ds(base + c1 * CHUNK, CHUNK)], ws1).wait()
            gather(c1 + 2, buf1, gs1)

        return ()

    lax.fori_loop(0, N_CHUNKS // 2, body, (), unroll=False)

    cL = N_CHUNKS - 2
    pltpu.make_async_copy(
        buf0, out_hbm.at[pl.ds(base + cL * CHUNK, CHUNK)], ws0).wait()
    pltpu.make_async_copy(
        buf1, out_hbm.at[pl.ds(base + (cL + 1) * CHUNK, CHUNK)], ws1).wait()


def kernel(x, embedding):
    idx = x.reshape(NUM_WORKERS, N_CHUNKS, CHUNK)
    out = _gather_kernel(idx, embedding)
    return out.reshape(B, S, H)


# speedup vs baseline: 7.7786x; 1.0637x over previous
import functools

import jax
import jax.numpy as jnp
from jax import lax
from jax.experimental import pallas as pl
from jax.experimental.pallas import tpu as pltpu
from jax.experimental.pallas import tpu_sc as plsc

B, S, H = 1024, 200, 128
N = B * S
NUM_WORKERS = 32
ROWS_PER_W = N // NUM_WORKERS
CHUNK = 128
N_CHUNKS = ROWS_PER_W // CHUNK
NBUF = 5

_mesh = plsc.VectorSubcoreMesh(core_axis_name="c", subcore_axis_name="s")


@functools.partial(
    pl.kernel,
    mesh=_mesh,
    out_type=jax.ShapeDtypeStruct((N, H), jnp.float32),
    scratch_types=(
        [pltpu.VMEM((N_CHUNKS, CHUNK), jnp.int32)]
        + [pltpu.VMEM((CHUNK, H), jnp.float32) for _ in range(NBUF)]
        + [pltpu.SemaphoreType.DMA for _ in range(2 * NBUF)]
    ),
)
def _gather_kernel(idx_hbm, table_hbm, out_hbm, idx_v, *rest):
    bufs = rest[:NBUF]
    gs = rest[NBUF:2 * NBUF]
    ws = rest[2 * NBUF:]
    wid = lax.axis_index("s") * 2 + lax.axis_index("c")
    base = wid * ROWS_PER_W
    pltpu.sync_copy(idx_hbm.at[wid], idx_v)

    def gather_desc(c, buf, sem):
        return pltpu.make_async_copy(table_hbm.at[idx_v.at[c]], buf, sem)

    def write_desc(c, buf, sem):
        return pltpu.make_async_copy(
            buf, out_hbm.at[pl.ds(base + c * CHUNK, CHUNK)], sem)

    for b in range(NBUF):
        gather_desc(b, bufs[b], gs[b]).start()

    def body(i, _):
        cbase = i * NBUF
        for b in range(NBUF):
            c = cbase + b
            gather_desc(c, bufs[b], gs[b]).wait()
            write_desc(c, bufs[b], ws[b]).start()
        for b in range(NBUF):
            c = cbase + b + NBUF

            @pl.when(c < N_CHUNKS)
            def _(c=c, b=b):
                write_desc(c - NBUF, bufs[b], ws[b]).wait()
                gather_desc(c, bufs[b], gs[b]).start()

        return ()

    lax.fori_loop(0, N_CHUNKS // NBUF, body, (), unroll=False)

    cL = N_CHUNKS - NBUF
    for b in range(NBUF):
        write_desc(cL + b, bufs[b], ws[b]).wait()


def kernel(x, embedding):
    idx = x.reshape(NUM_WORKERS, N_CHUNKS, CHUNK)
    out = _gather_kernel(idx, embedding)
    return out.reshape(B, S, H)


# speedup vs baseline: 7.8900x; 1.0143x over previous
import functools

import jax
import jax.numpy as jnp
from jax import lax
from jax.experimental import pallas as pl
from jax.experimental.pallas import tpu as pltpu
from jax.experimental.pallas import tpu_sc as plsc

B, S, H = 1024, 200, 128
N = B * S
NUM_WORKERS = 32
ROWS_PER_W = N // NUM_WORKERS
CHUNK = 64
N_CHUNKS = ROWS_PER_W // CHUNK
NBUF = 10

_mesh = plsc.VectorSubcoreMesh(core_axis_name="c", subcore_axis_name="s")


@functools.partial(
    pl.kernel,
    mesh=_mesh,
    out_type=jax.ShapeDtypeStruct((N, H), jnp.float32),
    scratch_types=(
        [pltpu.VMEM((N_CHUNKS, CHUNK), jnp.int32)]
        + [pltpu.VMEM((CHUNK, H), jnp.float32) for _ in range(NBUF)]
        + [pltpu.SemaphoreType.DMA for _ in range(2 * NBUF)]
    ),
)
def _gather_kernel(idx_hbm, table_hbm, out_hbm, idx_v, *rest):
    bufs = rest[:NBUF]
    gs = rest[NBUF:2 * NBUF]
    ws = rest[2 * NBUF:]
    wid = lax.axis_index("s") * 2 + lax.axis_index("c")
    base = wid * ROWS_PER_W
    pltpu.sync_copy(idx_hbm.at[wid], idx_v)

    def gather_desc(c, buf, sem):
        return pltpu.make_async_copy(table_hbm.at[idx_v.at[c]], buf, sem)

    def write_desc(c, buf, sem):
        return pltpu.make_async_copy(
            buf, out_hbm.at[pl.ds(base + c * CHUNK, CHUNK)], sem)

    for b in range(NBUF):
        gather_desc(b, bufs[b], gs[b]).start()

    def body(i, _):
        cbase = i * NBUF
        for b in range(NBUF):
            c = cbase + b
            gather_desc(c, bufs[b], gs[b]).wait()
            write_desc(c, bufs[b], ws[b]).start()
        for b in range(NBUF):
            c = cbase + b + NBUF

            @pl.when(c < N_CHUNKS)
            def _(c=c, b=b):
                write_desc(c - NBUF, bufs[b], ws[b]).wait()
                gather_desc(c, bufs[b], gs[b]).start()

        return ()

    lax.fori_loop(0, N_CHUNKS // NBUF, body, (), unroll=False)

    cL = N_CHUNKS - NBUF
    for b in range(NBUF):
        write_desc(cL + b, bufs[b], ws[b]).wait()


def kernel(x, embedding):
    idx = x.reshape(NUM_WORKERS, N_CHUNKS, CHUNK)
    out = _gather_kernel(idx, embedding)
    return out.reshape(B, S, H)
